# baseline (speedup 1.0000x reference)
import functools

import jax
import jax.numpy as jnp
from jax import lax
from jax.experimental import pallas as pl
from jax.experimental.pallas import tpu as pltpu
from jax.experimental.pallas import tpu_sc as plsc

N = 10000
E = 160000
HD = 256
H3 = 3 * HD
NG = 64
NSC = 2
NSUB = 16
CH = 128
EPAD = 163840
EPW = EPAD // NSUB
NP = 10240
RPW = NP // NSUB
DUMMY = 10016
BN = 1000
GRID = N // BN


NCH = EPW // CH
NBUF = 4


def _sc_agg(x2, gidx, dstp, zrows):
    mesh = plsc.VectorSubcoreMesh(core_axis_name="c", subcore_axis_name="s")

    @functools.partial(
        pl.kernel,
        out_type=jax.ShapeDtypeStruct((NSC, NP, 128), jnp.float32),
        mesh=mesh,
        scratch_types=[
            pltpu.VMEM((CH,), jnp.int32),
            pltpu.VMEM((CH,), jnp.int32),
            pltpu.VMEM((CH,), jnp.int32),
            pltpu.VMEM((CH,), jnp.int32),
            pltpu.VMEM((CH,), jnp.int32),
            pltpu.VMEM((CH,), jnp.int32),
            pltpu.VMEM((CH,), jnp.int32),
            pltpu.VMEM((CH,), jnp.int32),
            pltpu.VMEM((CH, 128), jnp.float32),
            pltpu.VMEM((CH, 128), jnp.float32),
            pltpu.VMEM_SHARED((NP, 128), jnp.float32),
            pltpu.SemaphoreType.DMA,
            pltpu.SemaphoreType.DMA,
            pltpu.SemaphoreType.DMA,
            pltpu.SemaphoreType.DMA,
            pltpu.SemaphoreType.DMA,
            pltpu.SemaphoreType.DMA,
            pltpu.SemaphoreType.DMA,
            pltpu.SemaphoreType.DMA,
        ],
    )
    def k(x2_hbm, gidx_hbm, dst_hbm, z_hbm, out_hbm,
          ig0, ig1, ig2, ig3, is0, is1, is2, is3, rows0, rows1, acc,
          sg0, sg1, ss0, ss1, si0, si1, si2, si3):
        igs = (ig0, ig1, ig2, ig3)
        iss = (is0, is1, is2, is3)
        rws = (rows0, rows1)
        sgs = (sg0, sg1)
        sss = (ss0, ss1)
        sis = (si0, si1, si2, si3)
        cid = lax.axis_index("c")
        sid = lax.axis_index("s")
        ebase = sid * EPW
        gbase = cid * EPAD + ebase

        def idx_load(c, q):
            pltpu.async_copy(gidx_hbm.at[pl.ds(gbase + c * CH, CH)],
                             igs[q], sis[q])
            pltpu.async_copy(dst_hbm.at[pl.ds(ebase + c * CH, CH)],
                             iss[q], sis[q])

        def idx_wait(c, q):
            pltpu.make_async_copy(gidx_hbm.at[pl.ds(gbase + c * CH, CH)],
                                  igs[q], sis[q]).wait()
            pltpu.make_async_copy(dst_hbm.at[pl.ds(ebase + c * CH, CH)],
                                  iss[q], sis[q]).wait()

        def gather(q, b):
            pltpu.async_copy(x2_hbm.at[igs[q]], rws[b], sgs[b])

        def gather_wait(q, b):
            pltpu.make_async_copy(x2_hbm.at[igs[q]], rws[b], sgs[b]).wait()

        def scat(q, b):
            pltpu.async_copy(rws[b], acc.at[iss[q]], sss[b], add=True)

        def scat_wait(q, b):
            pltpu.make_async_copy(rws[b], acc.at[iss[q]], sss[b]).wait()

        def step(c, ph, has_scat_wait=True, has_load=True, has_gather=True):
            b = ph % 2
            nb = 1 - b
            if has_gather:
                idx_wait(c + 1, (ph + 1) % 4)
            if has_scat_wait:
                scat_wait((ph - 1) % 4, nb)
            if has_gather:
                gather((ph + 1) % 4, nb)
            gather_wait(ph, b)
            scat(ph, b)
            if has_load:
                idx_load(c + 3, (ph + 3) % 4)

        idx_load(0, 0)
        idx_load(1, 1)
        idx_load(2, 2)
        pltpu.sync_copy(z_hbm, acc.at[pl.ds(sid * RPW, RPW)])
        plsc.subcore_barrier()
        idx_wait(0, 0)
        gather(0, 0)
        step(0, 0, has_scat_wait=False)
        step(1, 1)
        step(2, 2)

        def body(j, carry):
            c0 = 3 + j * 4
            for u in range(4):
                step(c0 + u, (3 + u) % 4)
            return carry

        lax.fori_loop(0, (NCH - 8) // 4, body, 0)
        step(NCH - 5, (NCH - 5) % 4)
        step(NCH - 4, (NCH - 4) % 4)
        step(NCH - 3, (NCH - 3) % 4, has_load=False)
        step(NCH - 2, (NCH - 2) % 4, has_load=False)
        step(NCH - 1, (NCH - 1) % 4, has_load=False, has_gather=False)
        scat_wait((NCH - 1) % 4, (NCH - 1) % 2)
        plsc.subcore_barrier()
        pltpu.sync_copy(acc.at[pl.ds(sid * RPW, RPW)],
                        out_hbm.at[cid, pl.ds(sid * RPW, RPW)])

    return k(x2, gidx, dstp, zrows)


def _row_spec(w):
    return pl.BlockSpec((BN, w), lambda i: (i, 0))


def _full_spec(h, w):
    return pl.BlockSpec((h, w), lambda i: (0, 0))


def _acc(ref, val, i):
    @pl.when(i == 0)
    def _():
        ref[...] = val

    @pl.when(i > 0)
    def _():
        ref[...] += val


def _stage_a(x, agg0, agg1, w1f, c1, w2c, b2c, p1h):
    def body(x_ref, a0_ref, a1_ref, w1_ref, c1_ref, w2_ref, b2_ref, p_ref,
             h2_ref, s_ref, cnt_ref):
        i = pl.program_id(0)
        h0 = x_ref[...] + jnp.concatenate([a0_ref[...], a1_ref[...]], axis=1)
        h1 = jnp.maximum(
            jnp.dot(h0, w1_ref[...], preferred_element_type=jnp.float32)
            + c1_ref[...], 0.0)
        parts = []
        for l in range(3):
            sl = slice(l * HD, (l + 1) * HD)
            parts.append(jnp.maximum(
                jnp.dot(h1[:, sl], w2_ref[:, sl],
                        preferred_element_type=jnp.float32)
                + b2_ref[:, sl], 0.0))
        h2 = jnp.concatenate(parts, axis=1)
        h2_ref[...] = h2
        pt = p_ref[...].T
        _acc(s_ref, jnp.dot(pt, h2, preferred_element_type=jnp.float32), i)
        _acc(cnt_ref,
             jnp.dot(pt, jnp.ones((BN, HD), jnp.float32),
                     preferred_element_type=jnp.float32), i)

    return pl.pallas_call(
        body,
        grid=(GRID,),
        in_specs=[
            _row_spec(HD),
            pl.BlockSpec((BN, 128), lambda i: (i, 0)),
            pl.BlockSpec((BN, 128), lambda i: (i, 0)),
            _full_spec(HD, H3),
            _full_spec(1, H3),
            _full_spec(HD, H3),
            _full_spec(1, H3),
            _row_spec(NG),
        ],
        out_specs=[_row_spec(H3), _full_spec(NG, H3), _full_spec(NG, HD)],
        out_shape=[jax.ShapeDtypeStruct((N, H3), jnp.float32),
                   jax.ShapeDtypeStruct((NG, H3), jnp.float32),
                   jax.ShapeDtypeStruct((NG, HD), jnp.float32)],
    )(x, agg0, agg1, w1f, c1, w2c, b2c, p1h)


def _stage_b(h2, s, cnt, p1h, gac):
    def body(h2_ref, s_ref, cnt_ref, p_ref, ga_ref, out_ref, v_ref):
        i = pl.program_id(0)
        c = jnp.maximum(cnt_ref[...], 1.0)
        cntc = jnp.concatenate([c, c, c], axis=1)
        mean = s_ref[...] / cntc
        mb = jnp.dot(p_ref[...], ga_ref[...] * mean,
                     preferred_element_type=jnp.float32)
        out = h2_ref[...] - mb
        out_ref[...] = out
        _acc(v_ref,
             jnp.dot(p_ref[...].T, out * out,
                     preferred_element_type=jnp.float32), i)

    return pl.pallas_call(
        body,
        grid=(GRID,),
        in_specs=[_row_spec(H3), _full_spec(NG, H3), _full_spec(NG, HD),
                  _row_spec(NG), _full_spec(1, H3)],
        out_specs=[_row_spec(H3), _full_spec(NG, H3)],
        out_shape=[jax.ShapeDtypeStruct((N, H3), jnp.float32),
                   jax.ShapeDtypeStruct((NG, H3), jnp.float32)],
    )(h2, s, cnt, p1h, gac)


def _stage_c(out, v, cnt, p1h, ggc, gbc):
    def body(out_ref, v_ref, cnt_ref, p_ref, gg_ref, gb_ref,
             emb_ref, pool_ref):
        i = pl.program_id(0)
        c = jnp.maximum(cnt_ref[...], 1.0)
        cntc = jnp.concatenate([c, c, c], axis=1)
        istd = lax.rsqrt(v_ref[...] / cntc + 1e-5)
        sb = jnp.dot(p_ref[...], istd, preferred_element_type=jnp.float32)
        emb = jnp.maximum(out_ref[...] * sb * gg_ref[...] + gb_ref[...], 0.0)
        emb_ref[...] = emb[:, 2 * HD:]
        _acc(pool_ref,
             jnp.dot(p_ref[...].T, emb,
                     preferred_element_type=jnp.float32), i)

    return pl.pallas_call(
        body,
        grid=(GRID,),
        in_specs=[_row_spec(H3), _full_spec(NG, H3), _full_spec(NG, HD),
                  _row_spec(NG), _full_spec(1, H3), _full_spec(1, H3)],
        out_specs=[_row_spec(HD), _full_spec(NG, H3)],
        out_shape=[jax.ShapeDtypeStruct((N, HD), jnp.float32),
                   jax.ShapeDtypeStruct((NG, H3), jnp.float32)],
    )(out, v, cnt, p1h, ggc, gbc)


def _head(pool, cnt, l1w, l1b, l2w, l2b):
    def body(pool_ref, cnt_ref, w1_ref, b1_ref, w2_ref, b2_ref, z_ref):
        c = jnp.maximum(cnt_ref[...], 1.0)
        ic = 1.0 / jnp.concatenate([c, c, c], axis=1)
        pooled = pool_ref[...] * ic
        z1 = jnp.maximum(
            jnp.dot(pooled, w1_ref[...], preferred_element_type=jnp.float32)
            + b1_ref[...], 0.0)
        z2 = (jnp.dot(z1, w2_ref[...], preferred_element_type=jnp.float32)
              + b2_ref[...])
        mask = lax.broadcasted_iota(jnp.int32, (NG, 128), 1) < 10
        zm = jnp.where(mask, z2, -1e30)
        m = jnp.max(zm, axis=1, keepdims=True)
        e = jnp.where(mask, jnp.exp(zm - m), 0.0)
        ssum = jnp.sum(e, axis=1, keepdims=True)
        z_ref[...] = zm - m - jnp.log(ssum)

    return pl.pallas_call(
        body,
        grid=(1,),
        in_specs=[_full_spec(NG, H3), _full_spec(NG, HD), _full_spec(H3, H3),
                  _full_spec(1, H3), _full_spec(H3, 128), _full_spec(1, 128)],
        out_specs=pl.BlockSpec((NG, 128), lambda i: (0, 0)),
        out_shape=jax.ShapeDtypeStruct((NG, 128), jnp.float32),
    )(pool, cnt, l1w, l1b, l2w, l2b)


def kernel(x, edge_index, batch, params):
    src = edge_index[0]
    dst = edge_index[1]
    pad = EPAD - E
    srcp = jnp.concatenate([src, jnp.zeros((pad,), jnp.int32)])
    dstp = jnp.concatenate([dst, jnp.full((pad,), DUMMY, jnp.int32)])
    gidx = jnp.concatenate([srcp * 2, srcp * 2 + 1])
    zrows = jnp.zeros((RPW, 128), jnp.float32)
    p1h = (batch[:, None] == jnp.arange(NG, dtype=jnp.int32)[None, :]
           ).astype(jnp.float32)

    bn_scale = 1.0 / jnp.sqrt(jnp.float32(1.0 + 1e-5))
    layers = params["layers"]
    w1f = jnp.concatenate(
        [lp["W1"] * (lp["bn_g"] * bn_scale)[None, :] for lp in layers], axis=1)
    c1 = jnp.concatenate(
        [lp["b1"] * lp["bn_g"] * bn_scale + lp["bn_b"] for lp in layers]
    )[None, :]
    w2c = jnp.concatenate([lp["W2"] for lp in layers], axis=1)
    b2c = jnp.concatenate([lp["b2"] for lp in layers])[None, :]
    gac = jnp.concatenate([lp["gn_a"] for lp in layers])[None, :]
    ggc = jnp.concatenate([lp["gn_g"] for lp in layers])[None, :]
    gbc = jnp.concatenate([lp["gn_b"] for lp in layers])[None, :]

    agg = _sc_agg(x.reshape(2 * N, 128), gidx, dstp, zrows)
    h2, s, cnt = _stage_a(x, agg[0], agg[1], w1f, c1, w2c, b2c, p1h)
    out, v = _stage_b(h2, s, cnt, p1h, gac)
    emb, pool = _stage_c(out, v, cnt, p1h, ggc, gbc)

    l2w = jnp.zeros((H3, 128), jnp.float32).at[:, :10].set(params["l2_W"])
    l2b = jnp.zeros((1, 128), jnp.float32).at[:, :10].set(params["l2_b"])
    zfull = _head(pool, cnt, params["l1_W"], params["l1_b"][None, :], l2w, l2b)
    return (emb, zfull[:, :10])


# speedup vs baseline: 1.0940x; 1.0940x over previous
import functools

import jax
import jax.numpy as jnp
from jax import lax
from jax.experimental import pallas as pl
from jax.experimental.pallas import tpu as pltpu
from jax.experimental.pallas import tpu_sc as plsc

N = 10000
E = 160000
HD = 256
H3 = 3 * HD
NG = 64
NSC = 2
NSUB = 16
CH = 128
EPAD = 163840
EPW = EPAD // NSUB
NP = 10240
RPW = NP // NSUB
DUMMY = 10016
BN = 1000
GRID = N // BN


NCH = EPW // CH
NBUF = 4


def _sc_agg(x2, gidx, dstp, zrows):
    mesh = plsc.VectorSubcoreMesh(core_axis_name="c", subcore_axis_name="s")

    @functools.partial(
        pl.kernel,
        out_type=jax.ShapeDtypeStruct((NSC, NP, 128), jnp.float32),
        mesh=mesh,
        scratch_types=[
            pltpu.VMEM((CH,), jnp.int32),
            pltpu.VMEM((CH,), jnp.int32),
            pltpu.VMEM((CH,), jnp.int32),
            pltpu.VMEM((CH,), jnp.int32),
            pltpu.VMEM((CH,), jnp.int32),
            pltpu.VMEM((CH,), jnp.int32),
            pltpu.VMEM((CH,), jnp.int32),
            pltpu.VMEM((CH,), jnp.int32),
            pltpu.VMEM((CH, 128), jnp.float32),
            pltpu.VMEM((CH, 128), jnp.float32),
            pltpu.VMEM_SHARED((NP, 128), jnp.float32),
            pltpu.SemaphoreType.DMA,
            pltpu.SemaphoreType.DMA,
            pltpu.SemaphoreType.DMA,
            pltpu.SemaphoreType.DMA,
            pltpu.SemaphoreType.DMA,
            pltpu.SemaphoreType.DMA,
            pltpu.SemaphoreType.DMA,
            pltpu.SemaphoreType.DMA,
        ],
    )
    def k(x2_hbm, gidx_hbm, dst_hbm, z_hbm, out_hbm,
          ig0, ig1, ig2, ig3, is0, is1, is2, is3, rows0, rows1, acc,
          sg0, sg1, ss0, ss1, si0, si1, si2, si3):
        igs = (ig0, ig1, ig2, ig3)
        iss = (is0, is1, is2, is3)
        rws = (rows0, rows1)
        sgs = (sg0, sg1)
        sss = (ss0, ss1)
        sis = (si0, si1, si2, si3)
        cid = lax.axis_index("c")
        sid = lax.axis_index("s")
        ebase = sid * EPW
        gbase = cid * EPAD + ebase

        def idx_load(c, q):
            pltpu.async_copy(gidx_hbm.at[pl.ds(gbase + c * CH, CH)],
                             igs[q], sis[q])
            pltpu.async_copy(dst_hbm.at[pl.ds(ebase + c * CH, CH)],
                             iss[q], sis[q])

        def idx_wait(c, q):
            pltpu.make_async_copy(gidx_hbm.at[pl.ds(gbase + c * CH, CH)],
                                  igs[q], sis[q]).wait()
            pltpu.make_async_copy(dst_hbm.at[pl.ds(ebase + c * CH, CH)],
                                  iss[q], sis[q]).wait()

        def gather(q, b):
            pltpu.async_copy(x2_hbm.at[igs[q]], rws[b], sgs[b])

        def gather_wait(q, b):
            pltpu.make_async_copy(x2_hbm.at[igs[q]], rws[b], sgs[b]).wait()

        def scat(q, b):
            pltpu.async_copy(rws[b], acc.at[iss[q]], sss[b], add=True)

        def scat_wait(q, b):
            pltpu.make_async_copy(rws[b], acc.at[iss[q]], sss[b]).wait()

        def step(c, ph, has_scat_wait=True, has_load=True, has_gather=True):
            b = ph % 2
            nb = 1 - b
            if has_gather:
                idx_wait(c + 1, (ph + 1) % 4)
            if has_scat_wait:
                scat_wait((ph - 1) % 4, nb)
            if has_gather:
                gather((ph + 1) % 4, nb)
            gather_wait(ph, b)
            scat(ph, b)
            if has_load:
                idx_load(c + 3, (ph + 3) % 4)

        idx_load(0, 0)
        idx_load(1, 1)
        idx_load(2, 2)
        pltpu.sync_copy(z_hbm, acc.at[pl.ds(sid * RPW, RPW)])
        plsc.subcore_barrier()
        idx_wait(0, 0)
        gather(0, 0)
        step(0, 0, has_scat_wait=False)
        step(1, 1)
        step(2, 2)

        def body(j, carry):
            c0 = 3 + j * 4
            for u in range(4):
                step(c0 + u, (3 + u) % 4)
            return carry

        lax.fori_loop(0, (NCH - 8) // 4, body, 0)
        step(NCH - 5, (NCH - 5) % 4)
        step(NCH - 4, (NCH - 4) % 4)
        step(NCH - 3, (NCH - 3) % 4, has_load=False)
        step(NCH - 2, (NCH - 2) % 4, has_load=False)
        step(NCH - 1, (NCH - 1) % 4, has_load=False, has_gather=False)
        scat_wait((NCH - 1) % 4, (NCH - 1) % 2)
        plsc.subcore_barrier()
        pltpu.sync_copy(acc.at[pl.ds(sid * RPW, RPW)],
                        out_hbm.at[cid, pl.ds(sid * RPW, RPW)])

    return k(x2, gidx, dstp, zrows)


def _row_spec(w):
    return pl.BlockSpec((BN, w), lambda i: (i, 0))


def _full_spec(h, w):
    return pl.BlockSpec((h, w), lambda i: (0, 0))


def _acc(ref, val, i):
    @pl.when(i == 0)
    def _():
        ref[...] = val

    @pl.when(i > 0)
    def _():
        ref[...] += val


def _pt_blk(b_ref):
    b2d = b_ref[0]
    return (lax.broadcasted_iota(jnp.int32, (NG, BN), 0) == b2d
            ).astype(jnp.float32)


def _stage_a(x, agg, w1f, c1, w2c, b2c, batch3):
    def body(x_ref, a0_ref, a1_ref, w1_ref, c1_ref, w2_ref, b2_ref, b_ref,
             h2_ref, s_ref, s2_ref, cnt_ref):
        i = pl.program_id(0)
        h0 = x_ref[...] + jnp.concatenate([a0_ref[0], a1_ref[0]], axis=1)
        h1 = jnp.maximum(
            jnp.dot(h0, w1_ref[...], preferred_element_type=jnp.float32)
            + c1_ref[...], 0.0)
        parts = []
        for l in range(3):
            sl = slice(l * HD, (l + 1) * HD)
            parts.append(jnp.maximum(
                jnp.dot(h1[:, sl], w2_ref[:, sl],
                        preferred_element_type=jnp.float32)
                + b2_ref[:, sl], 0.0))
        h2 = jnp.concatenate(parts, axis=1)
        h2_ref[...] = h2
        pt = _pt_blk(b_ref)
        _acc(s_ref, jnp.dot(pt, h2, preferred_element_type=jnp.float32), i)
        _acc(s2_ref,
             jnp.dot(pt, h2 * h2, preferred_element_type=jnp.float32), i)
        _acc(cnt_ref,
             jnp.dot(pt, jnp.ones((BN, HD), jnp.float32),
                     preferred_element_type=jnp.float32), i)

    return pl.pallas_call(
        body,
        grid=(GRID,),
        in_specs=[
            _row_spec(HD),
            pl.BlockSpec((1, BN, 128), lambda i: (0, i, 0)),
            pl.BlockSpec((1, BN, 128), lambda i: (1, i, 0)),
            _full_spec(HD, H3),
            _full_spec(1, H3),
            _full_spec(HD, H3),
            _full_spec(1, H3),
            pl.BlockSpec((1, 1, BN), lambda i: (i, 0, 0)),
        ],
        out_specs=[_row_spec(H3), _full_spec(NG, H3), _full_spec(NG, H3),
                   _full_spec(NG, HD)],
        out_shape=[jax.ShapeDtypeStruct((N, H3), jnp.float32),
                   jax.ShapeDtypeStruct((NG, H3), jnp.float32),
                   jax.ShapeDtypeStruct((NG, H3), jnp.float32),
                   jax.ShapeDtypeStruct((NG, HD), jnp.float32)],
    )(x, agg, agg, w1f, c1, w2c, b2c, batch3)


def _stage_b(h2, s, s2, cnt, batch3, gac, ggc, gbc):
    def body(h2_ref, s_ref, s2_ref, cnt_ref, b_ref, ga_ref, gg_ref, gb_ref,
             emb_ref, pool_ref):
        i = pl.program_id(0)
        c = jnp.maximum(cnt_ref[...], 1.0)
        cntc = jnp.concatenate([c, c, c], axis=1)
        a = ga_ref[...]
        mean = s_ref[...] / cntc
        var = s2_ref[...] / cntc - (2.0 * a - a * a) * mean * mean
        istd = lax.rsqrt(var + 1e-5)
        sg = istd * gg_ref[...]
        pt = _pt_blk(b_ref)
        b1 = jnp.dot(pt.T, sg, preferred_element_type=jnp.float32)
        b0 = jnp.dot(pt.T, a * mean * sg,
                     preferred_element_type=jnp.float32)
        emb = jnp.maximum(h2_ref[...] * b1 - b0 + gb_ref[...], 0.0)
        emb_ref[...] = emb[:, 2 * HD:]
        _acc(pool_ref,
             jnp.dot(pt, emb, preferred_element_type=jnp.float32), i)

    return pl.pallas_call(
        body,
        grid=(GRID,),
        in_specs=[_row_spec(H3), _full_spec(NG, H3), _full_spec(NG, H3),
                  _full_spec(NG, HD), pl.BlockSpec((1, 1, BN), lambda i: (i, 0, 0)),
                  _full_spec(1, H3), _full_spec(1, H3), _full_spec(1, H3)],
        out_specs=[_row_spec(HD), _full_spec(NG, H3)],
        out_shape=[jax.ShapeDtypeStruct((N, HD), jnp.float32),
                   jax.ShapeDtypeStruct((NG, H3), jnp.float32)],
    )(h2, s, s2, cnt, batch3, gac, ggc, gbc)


def _head(pool, cnt, l1w, l1b, l2w, l2b):
    def body(pool_ref, cnt_ref, w1_ref, b1_ref, w2_ref, b2_ref, z_ref):
        c = jnp.maximum(cnt_ref[...], 1.0)
        ic = 1.0 / jnp.concatenate([c, c, c], axis=1)
        pooled = pool_ref[...] * ic
        z1 = jnp.maximum(
            jnp.dot(pooled, w1_ref[...], preferred_element_type=jnp.float32)
            + b1_ref[...], 0.0)
        z2 = (jnp.dot(z1, w2_ref[...], preferred_element_type=jnp.float32)
              + b2_ref[...])
        mask = lax.broadcasted_iota(jnp.int32, (NG, 128), 1) < 10
        zm = jnp.where(mask, z2, -1e30)
        m = jnp.max(zm, axis=1, keepdims=True)
        e = jnp.where(mask, jnp.exp(zm - m), 0.0)
        ssum = jnp.sum(e, axis=1, keepdims=True)
        z_ref[...] = zm - m - jnp.log(ssum)

    return pl.pallas_call(
        body,
        grid=(1,),
        in_specs=[_full_spec(NG, H3), _full_spec(NG, HD), _full_spec(H3, H3),
                  _full_spec(1, H3), _full_spec(H3, 128), _full_spec(1, 128)],
        out_specs=pl.BlockSpec((NG, 128), lambda i: (0, 0)),
        out_shape=jax.ShapeDtypeStruct((NG, 128), jnp.float32),
    )(pool, cnt, l1w, l1b, l2w, l2b)


def kernel(x, edge_index, batch, params):
    src = edge_index[0]
    dst = edge_index[1]
    pad = EPAD - E
    srcp = jnp.concatenate([src, jnp.zeros((pad,), jnp.int32)])
    dstp = jnp.concatenate([dst, jnp.full((pad,), DUMMY, jnp.int32)])
    gidx = jnp.concatenate([srcp * 2, srcp * 2 + 1])
    zrows = jnp.zeros((RPW, 128), jnp.float32)
    batch3 = batch.reshape(GRID, 1, BN)

    bn_scale = 1.0 / jnp.sqrt(jnp.float32(1.0 + 1e-5))
    layers = params["layers"]
    w1f = jnp.concatenate(
        [lp["W1"] * (lp["bn_g"] * bn_scale)[None, :] for lp in layers], axis=1)
    c1 = jnp.concatenate(
        [lp["b1"] * lp["bn_g"] * bn_scale + lp["bn_b"] for lp in layers]
    )[None, :]
    w2c = jnp.concatenate([lp["W2"] for lp in layers], axis=1)
    b2c = jnp.concatenate([lp["b2"] for lp in layers])[None, :]
    gac = jnp.concatenate([lp["gn_a"] for lp in layers])[None, :]
    ggc = jnp.concatenate([lp["gn_g"] for lp in layers])[None, :]
    gbc = jnp.concatenate([lp["gn_b"] for lp in layers])[None, :]

    agg = _sc_agg(x.reshape(2 * N, 128), gidx, dstp, zrows)
    h2, sm, s2m, cnt = _stage_a(x, agg, w1f, c1, w2c, b2c, batch3)
    emb, pool = _stage_b(h2, sm, s2m, cnt, batch3, gac, ggc, gbc)

    l2w = jnp.zeros((H3, 128), jnp.float32).at[:, :10].set(params["l2_W"])
    l2b = jnp.zeros((1, 128), jnp.float32).at[:, :10].set(params["l2_b"])
    zfull = _head(pool, cnt, params["l1_W"], params["l1_b"][None, :], l2w, l2b)
    return (emb, zfull[:, :10])
